# baseline (speedup 1.0000x reference)
import functools

import jax
import jax.numpy as jnp
from jax import lax
from jax.experimental import pallas as pl
from jax.experimental.pallas import tpu as pltpu
from jax.experimental.pallas import tpu_sc as plsc

F = 32
NC = 2
NS = 16
NW = NC * NS
CH = 128
NSUB = 2
SUP = NSUB * CH


def _mm_bias_body(x_ref, w_ref, b_ref, o_ref):
    o_ref[...] = (
        jnp.dot(x_ref[...], w_ref[...], preferred_element_type=jnp.float32)
        + b_ref[...]
    )


def _mm_bias(x, w, b):
    m, k = x.shape
    f = w.shape[1]
    return pl.pallas_call(
        _mm_bias_body,
        grid=(1,),
        in_specs=[
            pl.BlockSpec((m, k), lambda i: (0, 0)),
            pl.BlockSpec((k, f), lambda i: (0, 0)),
            pl.BlockSpec((1, f), lambda i: (0, 0)),
        ],
        out_specs=pl.BlockSpec((m, f), lambda i: (0, 0)),
        out_shape=jax.ShapeDtypeStruct((m, f), jnp.float32),
    )(x, w, b)


def _edge_proj_body(e_ref, w1_ref, w2_ref, q1_ref, q2_ref):
    e = e_ref[...]
    n = e.shape[0] * 4 * F
    q1 = jnp.dot(e, w1_ref[...], preferred_element_type=jnp.float32)
    q2 = jnp.dot(e, w2_ref[...], preferred_element_type=jnp.float32)
    q1_ref[...] = q1.reshape(n)
    q2_ref[...] = q2.reshape(n)


def _edge_proj(e_packed, w1, w2, block):
    m, k = e_packed.shape
    out = jax.ShapeDtypeStruct((m * 4 * F,), jnp.float32)
    return pl.pallas_call(
        _edge_proj_body,
        grid=(m // block,),
        in_specs=[
            pl.BlockSpec((block, k), lambda i: (i, 0)),
            pl.BlockSpec((k, 4 * F), lambda i: (0, 0)),
            pl.BlockSpec((k, 4 * F), lambda i: (0, 0)),
        ],
        out_specs=[
            pl.BlockSpec((block * 4 * F,), lambda i: (i,)),
            pl.BlockSpec((block * 4 * F,), lambda i: (i,)),
        ],
        out_shape=[out, out],
    )(e_packed, w1, w2)


def _update_body(x_ref, a0_ref, a1_ref, wx_ref, wa_ref, bu_ref,
                 wm_ref, bm_ref, y_ref, p_ref):
    agg = a0_ref[0] + a1_ref[0]
    y = (
        jnp.dot(x_ref[...], wx_ref[...], preferred_element_type=jnp.float32)
        + jnp.dot(agg, wa_ref[...], preferred_element_type=jnp.float32)
        + bu_ref[...]
    )
    y_ref[...] = y
    p_ref[...] = (
        jnp.dot(y, wm_ref[...], preferred_element_type=jnp.float32)
        + bm_ref[...]
    )


def _update_and_project(x, aggp, wx, wa, bu, wm, bm):
    m, k = x.shape
    out = jax.ShapeDtypeStruct((m, k), jnp.float32)
    return pl.pallas_call(
        _update_body,
        grid=(1,),
        in_specs=[
            pl.BlockSpec((m, k), lambda i: (0, 0)),
            pl.BlockSpec((1, m, k), lambda i: (0, 0, 0)),
            pl.BlockSpec((1, m, k), lambda i: (1, 0, 0)),
            pl.BlockSpec((k, k), lambda i: (0, 0)),
            pl.BlockSpec((k, k), lambda i: (0, 0)),
            pl.BlockSpec((1, k), lambda i: (0, 0)),
            pl.BlockSpec((k, k), lambda i: (0, 0)),
            pl.BlockSpec((1, k), lambda i: (0, 0)),
        ],
        out_specs=[
            pl.BlockSpec((m, k), lambda i: (0, 0)),
            pl.BlockSpec((m, k), lambda i: (0, 0)),
        ],
        out_shape=[out, out],
    )(x, aggp, aggp, wx, wa, bu, wm, bm)


def _final_body(y_ref, a0_ref, a1_ref, oh0_ref, oh1_ref, oh2_ref, oh3_ref,
                wx_ref, wa_ref, bu_ref, wh_ref, bh_ref, wo_ref, bo_ref,
                o_ref):
    agg = a0_ref[0] + a1_ref[0]
    y2 = (
        jnp.dot(y_ref[...], wx_ref[...], preferred_element_type=jnp.float32)
        + jnp.dot(agg, wa_ref[...], preferred_element_type=jnp.float32)
        + bu_ref[...]
    )
    ohs = (oh0_ref, oh1_ref, oh2_ref, oh3_ref)
    pooled = jnp.zeros((64, F), jnp.float32)
    for k in range(4):
        pooled += lax.dot_general(
            ohs[k][...], y2[:, k * F:(k + 1) * F],
            (((0,), (0,)), ((), ())), preferred_element_type=jnp.float32)
    h = jnp.maximum(
        jnp.dot(pooled, wh_ref[...], preferred_element_type=jnp.float32)
        + bh_ref[...], 0.0)
    o_ref[...] = (
        jnp.dot(h, wo_ref[...], preferred_element_type=jnp.float32)
        + bo_ref[...]
    )


def _final(y, aggp, oh, wx, wa, bu, wh, bh, wo, bo):
    m, k = y.shape
    return pl.pallas_call(
        _final_body,
        grid=(1,),
        in_specs=[
            pl.BlockSpec((m, k), lambda i: (0, 0)),
            pl.BlockSpec((1, m, k), lambda i: (0, 0, 0)),
            pl.BlockSpec((1, m, k), lambda i: (1, 0, 0)),
            pl.BlockSpec((m, 64), lambda i: (0, 0)),
            pl.BlockSpec((m, 64), lambda i: (0, 0)),
            pl.BlockSpec((m, 64), lambda i: (0, 0)),
            pl.BlockSpec((m, 64), lambda i: (0, 0)),
            pl.BlockSpec((k, k), lambda i: (0, 0)),
            pl.BlockSpec((k, k), lambda i: (0, 0)),
            pl.BlockSpec((1, k), lambda i: (0, 0)),
            pl.BlockSpec((F, F), lambda i: (0, 0)),
            pl.BlockSpec((1, F), lambda i: (0, 0)),
            pl.BlockSpec((F, 1), lambda i: (0, 0)),
            pl.BlockSpec((1, 1), lambda i: (0, 0)),
        ],
        out_specs=pl.BlockSpec((64, 1), lambda i: (0, 0)),
        out_shape=jax.ShapeDtypeStruct((64, 1), jnp.float32),
    )(y, aggp, aggp, oh[0], oh[1], oh[2], oh[3],
      wx, wa, bu, wh, bh, wo, bo)


def _make_edge_pass(n_nodes, n_pad, n_edges):
    n_sup = n_edges // SUP
    m_sup = -(-n_sup // NW)
    kloop = (m_sup + 5) // 3
    zb = 80
    n_zb = n_nodes // zb
    zmax = -(-n_zb // NS)

    mesh = plsc.VectorSubcoreMesh(core_axis_name="c", subcore_axis_name="s")

    @functools.partial(
        pl.kernel,
        out_type=jax.ShapeDtypeStruct((2 * n_pad, F), jnp.float32),
        mesh=mesh,
        scratch_types=[
            pltpu.VMEM((SUP,), jnp.int32),
            pltpu.VMEM((SUP,), jnp.int32),
            pltpu.VMEM((SUP,), jnp.int32),
            pltpu.VMEM((NSUB, CH), jnp.int32),
            pltpu.VMEM((NSUB, CH), jnp.int32),
            pltpu.VMEM((NSUB, CH), jnp.int32),
            pltpu.VMEM((SUP, F), jnp.float32),
            pltpu.VMEM((SUP, F), jnp.float32),
            pltpu.VMEM((SUP, F), jnp.float32),
            pltpu.VMEM((zb, F), jnp.float32),
            pltpu.VMEM_SHARED((n_nodes, F), jnp.float32),
            pltpu.SemaphoreType.DMA,
            pltpu.SemaphoreType.DMA,
            pltpu.SemaphoreType.DMA,
            pltpu.SemaphoreType.DMA,
            pltpu.SemaphoreType.DMA,
            pltpu.SemaphoreType.DMA,
            pltpu.SemaphoreType.DMA,
            pltpu.SemaphoreType.DMA,
            pltpu.SemaphoreType.DMA,
        ],
        compiler_params=pltpu.CompilerParams(use_tc_tiling_on_sc=False),
    )
    def edge_pass(p_hbm, q_hbm, src_hbm, dst_hbm, out_hbm,
                  sv0, sv1, sv2, dv0, dv1, dv2, gv0, gv1, gv2, z_v, acc_sh,
                  ls0, ls1, ls2, gs0, gs1, gs2, ss0, ss1, ss2):
        c = lax.axis_index("c")
        s = lax.axis_index("s")
        wid = s * NC + c
        sv = (sv0, sv1, sv2)
        dv = (dv0, dv1, dv2)
        gv = (gv0, gv1, gv2)
        ls = (ls0, ls1, ls2)
        gs = (gs0, gs1, gs2)
        ss = (ss0, ss1, ss2)

        def zfill(j, _):
            z_v[j, pl.ds(0, 16)] = jnp.zeros((16,), jnp.float32)
            z_v[j, pl.ds(16, 16)] = jnp.zeros((16,), jnp.float32)
            return 0
        lax.fori_loop(0, zb, zfill, 0)

        def zcopy(i, _):
            blk = s + i * NS

            @pl.when(blk < n_zb)
            def _():
                pltpu.sync_copy(z_v, acc_sh.at[pl.ds(blk * zb, zb)])
            return 0
        lax.fori_loop(0, zmax, zcopy, 0)
        plsc.subcore_barrier()

        def lin_cps(sup, b):
            base = sup * SUP
            cps = [
                pltpu.make_async_copy(
                    src_hbm.at[pl.ds(base, SUP)], sv[b], ls[b]),
                pltpu.make_async_copy(
                    q_hbm.at[pl.ds(base, SUP)], gv[b], ls[b]),
            ]
            for j in range(NSUB):
                cps.append(pltpu.make_async_copy(
                    dst_hbm.at[pl.ds(base + j * CH, CH)], dv[b].at[j], ls[b]))
            return cps

        def g_cps(b):
            return [
                pltpu.make_async_copy(
                    p_hbm.at[sv[b].at[pl.ds(j * CH, CH)]],
                    gv[b].at[pl.ds(j * CH, CH)], gs[b])
                for j in range(NSUB)
            ]

        def s_cps(b):
            return [
                pltpu.make_async_copy(
                    gv[b].at[pl.ds(j * CH, CH)],
                    acc_sh.at[dv[b].at[j]], ss[b])
                for j in range(NSUB)
            ]

        def stage(i, a):
            b = (a + 2) % 3
            nxt = (a + 1) % 3
            sup_a = wid + i * NW

            @pl.when(sup_a < n_sup)
            def _():
                for cp in lin_cps(sup_a, a):
                    cp.wait()
                for cp in g_cps(a):
                    cp.start(add=True)

            sup_b = wid + (i - 1) * NW

            @pl.when(jnp.logical_and(i >= 1, sup_b < n_sup))
            def _():
                for cp in g_cps(b):
                    cp.wait()

                def relu_body(j, _):
                    for h in range(2):
                        sl = pl.ds(h * 16, 16)
                        gv[b][j, sl] = jnp.maximum(gv[b][j, sl], 0.0)
                    return 0
                lax.fori_loop(0, SUP, relu_body, 0, unroll=8)
                for cp in s_cps(b):
                    cp.start(add=True)

            sup_d = wid + (i - 2) * NW

            @pl.when(jnp.logical_and(i >= 2, sup_d < n_sup))
            def _():
                for cp in s_cps(nxt):
                    cp.wait()

            sup_c = wid + (i + 1) * NW

            @pl.when(sup_c < n_sup)
            def _():
                for cp in lin_cps(sup_c, nxt):
                    cp.start()

        @pl.when(wid < n_sup)
        def _():
            for cp in lin_cps(wid, 0):
                cp.start()

        def loop_body(k, _):
            stage(3 * k, 0)
            stage(3 * k + 1, 1)
            stage(3 * k + 2, 2)
            return 0
        lax.fori_loop(0, kloop, loop_body, 0)
        plsc.subcore_barrier()

        if n_pad > n_nodes:
            @pl.when(s == 0)
            def _():
                pltpu.sync_copy(
                    z_v.at[pl.ds(0, n_pad - n_nodes)],
                    out_hbm.at[pl.ds(c * n_pad + n_nodes, n_pad - n_nodes)])

        def fcopy(i, _):
            blk = s + i * NS

            @pl.when(blk < n_zb)
            def _():
                row0 = blk * zb
                pltpu.sync_copy(acc_sh.at[pl.ds(row0, zb)], z_v)
                pltpu.sync_copy(z_v, out_hbm.at[pl.ds(c * n_pad + row0, zb)])
            return 0
        lax.fori_loop(0, zmax, fcopy, 0)

    return edge_pass


def kernel(node_features, edge_features, edge_idx, batch_idx,
           WM1, bM1, WU1, bU1, WM2, bM2, WU2, bU2, Wh, bh, Wo, bo):
    n = node_features.shape[0]
    n_pad = -(-n // 32) * 32
    m4 = n_pad // 4

    eye4 = jnp.eye(4, dtype=jnp.float32)

    def bd4(w):
        return jnp.kron(eye4, w)

    def tile4(b):
        return jnp.tile(b, 4)[None, :]

    xp = jnp.pad(node_features, ((0, n_pad - n), (0, F - 21)))
    x_packed = xp.reshape(m4, 4 * F)

    n_edges = edge_features.shape[0]
    e_packed = edge_features.reshape(n_edges // 4, 12)
    src = edge_idx[0]
    dst = edge_idx[1]

    bfull = jnp.pad(batch_idx, (0, n_pad - n), constant_values=64)
    oh = tuple(
        (bfull[k::4][:, None] == jnp.arange(64)[None, :]).astype(jnp.float32)
        for k in range(4))

    wm1x = jnp.pad(WM1[:21], ((0, 11), (0, 0)))
    wu1x = jnp.pad(WU1[:21], ((0, 11), (0, 0)))

    edge_pass = _make_edge_pass(n, n_pad, n_edges)

    p1 = _mm_bias(x_packed, bd4(wm1x), tile4(bM1))
    q1p, q2p = _edge_proj(
        e_packed, jnp.kron(eye4, WM1[21:24]), jnp.kron(eye4, WM2[32:35]),
        block=8000)
    agg1 = edge_pass(p1.reshape(n_pad, F), q1p.reshape(n_edges, F), src, dst)
    y, p2 = _update_and_project(
        x_packed, agg1.reshape(2, m4, 4 * F), bd4(wu1x), bd4(WU1[21:53]),
        tile4(bU1), bd4(WM2[:32]), tile4(bM2))

    agg2 = edge_pass(p2.reshape(n_pad, F), q2p.reshape(n_edges, F), src, dst)

    return _final(
        y, agg2.reshape(2, m4, 4 * F), oh, bd4(WU2[:32]), bd4(WU2[32:64]),
        tile4(bU2), Wh, bh[None, :], Wo, bo[None, :])


# speedup vs baseline: 2.7606x; 2.7606x over previous
import functools

import jax
import jax.numpy as jnp
from jax import lax
from jax.experimental import pallas as pl
from jax.experimental.pallas import tpu as pltpu
from jax.experimental.pallas import tpu_sc as plsc

F = 32
NC = 2
NS = 16
NW = NC * NS
CH = 128
NSUB = 2
SUP = NSUB * CH


def _mm_bias_body(x_ref, w_ref, b_ref, o_ref):
    o_ref[...] = (
        jnp.dot(x_ref[...], w_ref[...], preferred_element_type=jnp.float32)
        + b_ref[...]
    )


def _mm_bias(x, w, b):
    m, k = x.shape
    f = w.shape[1]
    return pl.pallas_call(
        _mm_bias_body,
        grid=(1,),
        in_specs=[
            pl.BlockSpec((m, k), lambda i: (0, 0)),
            pl.BlockSpec((k, f), lambda i: (0, 0)),
            pl.BlockSpec((1, f), lambda i: (0, 0)),
        ],
        out_specs=pl.BlockSpec((m, f), lambda i: (0, 0)),
        out_shape=jax.ShapeDtypeStruct((m, f), jnp.float32),
    )(x, w, b)


def _edge_proj_body(et_ref, w1_ref, w2_ref, q1_ref, q2_ref):
    et = et_ref[...]
    dn = (((0,), (0,)), ((), ()))
    q1_ref[...] = lax.dot_general(
        et, w1_ref[...], dn, preferred_element_type=jnp.float32)
    q2_ref[...] = lax.dot_general(
        et, w2_ref[...], dn, preferred_element_type=jnp.float32)


def _edge_proj(et, w1, w2, block):
    k, m = et.shape
    out = jax.ShapeDtypeStruct((m, F), jnp.float32)
    return pl.pallas_call(
        _edge_proj_body,
        grid=(m // block,),
        in_specs=[
            pl.BlockSpec((k, block), lambda i: (0, i)),
            pl.BlockSpec((k, F), lambda i: (0, 0)),
            pl.BlockSpec((k, F), lambda i: (0, 0)),
        ],
        out_specs=[
            pl.BlockSpec((block, F), lambda i: (i, 0)),
            pl.BlockSpec((block, F), lambda i: (i, 0)),
        ],
        out_shape=[out, out],
    )(et, w1, w2)


def _update_body(x_ref, a0_ref, a1_ref, wx_ref, wa_ref, bu_ref,
                 wm_ref, bm_ref, y_ref, p_ref):
    agg = a0_ref[0] + a1_ref[0]
    y = (
        jnp.dot(x_ref[...], wx_ref[...], preferred_element_type=jnp.float32)
        + jnp.dot(agg, wa_ref[...], preferred_element_type=jnp.float32)
        + bu_ref[...]
    )
    y_ref[...] = y
    p_ref[...] = (
        jnp.dot(y, wm_ref[...], preferred_element_type=jnp.float32)
        + bm_ref[...]
    )


def _update_and_project(x, aggp, wx, wa, bu, wm, bm):
    m, k = x.shape
    out = jax.ShapeDtypeStruct((m, k), jnp.float32)
    return pl.pallas_call(
        _update_body,
        grid=(1,),
        in_specs=[
            pl.BlockSpec((m, k), lambda i: (0, 0)),
            pl.BlockSpec((1, m, k), lambda i: (0, 0, 0)),
            pl.BlockSpec((1, m, k), lambda i: (1, 0, 0)),
            pl.BlockSpec((k, k), lambda i: (0, 0)),
            pl.BlockSpec((k, k), lambda i: (0, 0)),
            pl.BlockSpec((1, k), lambda i: (0, 0)),
            pl.BlockSpec((k, k), lambda i: (0, 0)),
            pl.BlockSpec((1, k), lambda i: (0, 0)),
        ],
        out_specs=[
            pl.BlockSpec((m, k), lambda i: (0, 0)),
            pl.BlockSpec((m, k), lambda i: (0, 0)),
        ],
        out_shape=[out, out],
    )(x, aggp, aggp, wx, wa, bu, wm, bm)


def _final_body(y_ref, a0_ref, a1_ref, oh0_ref, oh1_ref, oh2_ref, oh3_ref,
                wx_ref, wa_ref, bu_ref, wh_ref, bh_ref, wo_ref, bo_ref,
                o_ref):
    agg = a0_ref[0] + a1_ref[0]
    y2 = (
        jnp.dot(y_ref[...], wx_ref[...], preferred_element_type=jnp.float32)
        + jnp.dot(agg, wa_ref[...], preferred_element_type=jnp.float32)
        + bu_ref[...]
    )
    ohs = (oh0_ref, oh1_ref, oh2_ref, oh3_ref)
    pooled = jnp.zeros((64, F), jnp.float32)
    for k in range(4):
        pooled += lax.dot_general(
            ohs[k][...], y2[:, k * F:(k + 1) * F],
            (((0,), (0,)), ((), ())), preferred_element_type=jnp.float32)
    h = jnp.maximum(
        jnp.dot(pooled, wh_ref[...], preferred_element_type=jnp.float32)
        + bh_ref[...], 0.0)
    o_ref[...] = (
        jnp.dot(h, wo_ref[...], preferred_element_type=jnp.float32)
        + bo_ref[...]
    )


def _final(y, aggp, oh, wx, wa, bu, wh, bh, wo, bo):
    m, k = y.shape
    return pl.pallas_call(
        _final_body,
        grid=(1,),
        in_specs=[
            pl.BlockSpec((m, k), lambda i: (0, 0)),
            pl.BlockSpec((1, m, k), lambda i: (0, 0, 0)),
            pl.BlockSpec((1, m, k), lambda i: (1, 0, 0)),
            pl.BlockSpec((m, 64), lambda i: (0, 0)),
            pl.BlockSpec((m, 64), lambda i: (0, 0)),
            pl.BlockSpec((m, 64), lambda i: (0, 0)),
            pl.BlockSpec((m, 64), lambda i: (0, 0)),
            pl.BlockSpec((k, k), lambda i: (0, 0)),
            pl.BlockSpec((k, k), lambda i: (0, 0)),
            pl.BlockSpec((1, k), lambda i: (0, 0)),
            pl.BlockSpec((F, F), lambda i: (0, 0)),
            pl.BlockSpec((1, F), lambda i: (0, 0)),
            pl.BlockSpec((F, 1), lambda i: (0, 0)),
            pl.BlockSpec((1, 1), lambda i: (0, 0)),
        ],
        out_specs=pl.BlockSpec((64, 1), lambda i: (0, 0)),
        out_shape=jax.ShapeDtypeStruct((64, 1), jnp.float32),
    )(y, aggp, aggp, oh[0], oh[1], oh[2], oh[3],
      wx, wa, bu, wh, bh, wo, bo)


def _make_edge_pass(n_nodes, n_pad, n_edges):
    n_sup = n_edges // SUP
    m_sup = -(-n_sup // NW)
    kloop = (m_sup + 5) // 3
    zb = 80
    n_zb = n_nodes // zb
    zmax = -(-n_zb // NS)

    mesh = plsc.VectorSubcoreMesh(core_axis_name="c", subcore_axis_name="s")

    @functools.partial(
        pl.kernel,
        out_type=jax.ShapeDtypeStruct((2 * n_pad, F), jnp.float32),
        mesh=mesh,
        scratch_types=[
            pltpu.VMEM((SUP,), jnp.int32),
            pltpu.VMEM((SUP,), jnp.int32),
            pltpu.VMEM((SUP,), jnp.int32),
            pltpu.VMEM((NSUB, CH), jnp.int32),
            pltpu.VMEM((NSUB, CH), jnp.int32),
            pltpu.VMEM((NSUB, CH), jnp.int32),
            pltpu.VMEM((SUP, F), jnp.float32),
            pltpu.VMEM((SUP, F), jnp.float32),
            pltpu.VMEM((SUP, F), jnp.float32),
            pltpu.VMEM((zb, F), jnp.float32),
            pltpu.VMEM_SHARED((n_nodes, F), jnp.float32),
            pltpu.SemaphoreType.DMA,
            pltpu.SemaphoreType.DMA,
            pltpu.SemaphoreType.DMA,
            pltpu.SemaphoreType.DMA,
            pltpu.SemaphoreType.DMA,
            pltpu.SemaphoreType.DMA,
            pltpu.SemaphoreType.DMA,
            pltpu.SemaphoreType.DMA,
            pltpu.SemaphoreType.DMA,
        ],
        compiler_params=pltpu.CompilerParams(use_tc_tiling_on_sc=False),
    )
    def edge_pass(p_hbm, q_hbm, src_hbm, dst_hbm, out_hbm,
                  sv0, sv1, sv2, dv0, dv1, dv2, gv0, gv1, gv2, z_v, acc_sh,
                  ls0, ls1, ls2, gs0, gs1, gs2, ss0, ss1, ss2):
        c = lax.axis_index("c")
        s = lax.axis_index("s")
        wid = s * NC + c
        sv = (sv0, sv1, sv2)
        dv = (dv0, dv1, dv2)
        gv = (gv0, gv1, gv2)
        ls = (ls0, ls1, ls2)
        gs = (gs0, gs1, gs2)
        ss = (ss0, ss1, ss2)

        def zfill(j, _):
            z_v[j, pl.ds(0, 16)] = jnp.zeros((16,), jnp.float32)
            z_v[j, pl.ds(16, 16)] = jnp.zeros((16,), jnp.float32)
            return 0
        lax.fori_loop(0, zb, zfill, 0)

        def zcopy(i, _):
            blk = s + i * NS

            @pl.when(blk < n_zb)
            def _():
                pltpu.sync_copy(z_v, acc_sh.at[pl.ds(blk * zb, zb)])
            return 0
        lax.fori_loop(0, zmax, zcopy, 0)
        plsc.subcore_barrier()

        def lin_cps(sup, b):
            base = sup * SUP
            cps = [
                pltpu.make_async_copy(
                    src_hbm.at[pl.ds(base, SUP)], sv[b], ls[b]),
                pltpu.make_async_copy(
                    q_hbm.at[pl.ds(base, SUP)], gv[b], ls[b]),
            ]
            for j in range(NSUB):
                cps.append(pltpu.make_async_copy(
                    dst_hbm.at[pl.ds(base + j * CH, CH)], dv[b].at[j], ls[b]))
            return cps

        def g_cps(b):
            return [
                pltpu.make_async_copy(
                    p_hbm.at[sv[b].at[pl.ds(j * CH, CH)]],
                    gv[b].at[pl.ds(j * CH, CH)], gs[b])
                for j in range(NSUB)
            ]

        def s_cps(b):
            return [
                pltpu.make_async_copy(
                    gv[b].at[pl.ds(j * CH, CH)],
                    acc_sh.at[dv[b].at[j]], ss[b])
                for j in range(NSUB)
            ]

        def stage(i, a):
            b = (a + 2) % 3
            nxt = (a + 1) % 3
            sup_a = wid + i * NW

            @pl.when(sup_a < n_sup)
            def _():
                for cp in lin_cps(sup_a, a):
                    cp.wait()
                for cp in g_cps(a):
                    cp.start(add=True)

            sup_b = wid + (i - 1) * NW

            @pl.when(jnp.logical_and(i >= 1, sup_b < n_sup))
            def _():
                for cp in g_cps(b):
                    cp.wait()

                def relu_body(j, _):
                    for h in range(2):
                        sl = pl.ds(h * 16, 16)
                        gv[b][j, sl] = jnp.maximum(gv[b][j, sl], 0.0)
                    return 0
                lax.fori_loop(0, SUP, relu_body, 0, unroll=8)
                for cp in s_cps(b):
                    cp.start(add=True)

            sup_d = wid + (i - 2) * NW

            @pl.when(jnp.logical_and(i >= 2, sup_d < n_sup))
            def _():
                for cp in s_cps(nxt):
                    cp.wait()

            sup_c = wid + (i + 1) * NW

            @pl.when(sup_c < n_sup)
            def _():
                for cp in lin_cps(sup_c, nxt):
                    cp.start()

        @pl.when(wid < n_sup)
        def _():
            for cp in lin_cps(wid, 0):
                cp.start()

        def loop_body(k, _):
            stage(3 * k, 0)
            stage(3 * k + 1, 1)
            stage(3 * k + 2, 2)
            return 0
        lax.fori_loop(0, kloop, loop_body, 0)
        plsc.subcore_barrier()

        if n_pad > n_nodes:
            @pl.when(s == 0)
            def _():
                pltpu.sync_copy(
                    z_v.at[pl.ds(0, n_pad - n_nodes)],
                    out_hbm.at[pl.ds(c * n_pad + n_nodes, n_pad - n_nodes)])

        def fcopy(i, _):
            blk = s + i * NS

            @pl.when(blk < n_zb)
            def _():
                row0 = blk * zb
                pltpu.sync_copy(acc_sh.at[pl.ds(row0, zb)], z_v)
                pltpu.sync_copy(z_v, out_hbm.at[pl.ds(c * n_pad + row0, zb)])
            return 0
        lax.fori_loop(0, zmax, fcopy, 0)

    return edge_pass


def kernel(node_features, edge_features, edge_idx, batch_idx,
           WM1, bM1, WU1, bU1, WM2, bM2, WU2, bU2, Wh, bh, Wo, bo):
    n = node_features.shape[0]
    n_pad = -(-n // 32) * 32
    m4 = n_pad // 4

    eye4 = jnp.eye(4, dtype=jnp.float32)

    def bd4(w):
        return jnp.kron(eye4, w)

    def tile4(b):
        return jnp.tile(b, 4)[None, :]

    xp = jnp.pad(node_features, ((0, n_pad - n), (0, F - 21)))
    x_packed = xp.reshape(m4, 4 * F)

    n_edges = edge_features.shape[0]
    src = edge_idx[0]
    dst = edge_idx[1]

    bfull = jnp.pad(batch_idx, (0, n_pad - n), constant_values=64)
    oh = tuple(
        (bfull[k::4][:, None] == jnp.arange(64)[None, :]).astype(jnp.float32)
        for k in range(4))

    wm1x = jnp.pad(WM1[:21], ((0, 11), (0, 0)))
    wu1x = jnp.pad(WU1[:21], ((0, 11), (0, 0)))

    edge_pass = _make_edge_pass(n, n_pad, n_edges)

    p1 = _mm_bias(x_packed, bd4(wm1x), tile4(bM1))
    q1, q2 = _edge_proj(
        edge_features.T, WM1[21:24], WM2[32:35], block=16000)
    agg1 = edge_pass(p1.reshape(n_pad, F), q1, src, dst)
    y, p2 = _update_and_project(
        x_packed, agg1.reshape(2, m4, 4 * F), bd4(wu1x), bd4(WU1[21:53]),
        tile4(bU1), bd4(WM2[:32]), tile4(bM2))

    agg2 = edge_pass(p2.reshape(n_pad, F), q2, src, dst)

    return _final(
        y, agg2.reshape(2, m4, 4 * F), oh, bd4(WU2[:32]), bd4(WU2[32:64]),
        tile4(bU2), Wh, bh[None, :], Wo, bo[None, :])


# speedup vs baseline: 2.7702x; 1.0035x over previous
import functools

import jax
import jax.numpy as jnp
from jax import lax
from jax.experimental import pallas as pl
from jax.experimental.pallas import tpu as pltpu
from jax.experimental.pallas import tpu_sc as plsc

F = 32
NC = 2
NS = 16
NW = NC * NS
CH = 128
NSUB = 2
SUP = NSUB * CH


def _mm_bias_body(x_ref, w_ref, b_ref, o_ref):
    o_ref[...] = (
        jnp.dot(x_ref[...], w_ref[...], preferred_element_type=jnp.float32)
        + b_ref[...]
    )


def _mm_bias(x, w, b):
    m, k = x.shape
    f = w.shape[1]
    return pl.pallas_call(
        _mm_bias_body,
        grid=(1,),
        in_specs=[
            pl.BlockSpec((m, k), lambda i: (0, 0)),
            pl.BlockSpec((k, f), lambda i: (0, 0)),
            pl.BlockSpec((1, f), lambda i: (0, 0)),
        ],
        out_specs=pl.BlockSpec((m, f), lambda i: (0, 0)),
        out_shape=jax.ShapeDtypeStruct((m, f), jnp.float32),
    )(x, w, b)


def _edge_proj_body(et_ref, w1_ref, w2_ref, q1_ref, q2_ref):
    et = et_ref[...]
    dn = (((0,), (0,)), ((), ()))
    q1_ref[...] = lax.dot_general(
        et, w1_ref[...], dn, preferred_element_type=jnp.float32)
    q2_ref[...] = lax.dot_general(
        et, w2_ref[...], dn, preferred_element_type=jnp.float32)


def _edge_proj(et, w1, w2, block):
    k, m = et.shape
    out = jax.ShapeDtypeStruct((m, F), jnp.float32)
    return pl.pallas_call(
        _edge_proj_body,
        grid=(m // block,),
        in_specs=[
            pl.BlockSpec((k, block), lambda i: (0, i)),
            pl.BlockSpec((k, F), lambda i: (0, 0)),
            pl.BlockSpec((k, F), lambda i: (0, 0)),
        ],
        out_specs=[
            pl.BlockSpec((block, F), lambda i: (i, 0)),
            pl.BlockSpec((block, F), lambda i: (i, 0)),
        ],
        out_shape=[out, out],
        compiler_params=pltpu.CompilerParams(
            fuse_transposed_lhs_in_matmul=True),
    )(et, w1, w2)


def _update_body(x_ref, a0_ref, a1_ref, wx_ref, wa_ref, bu_ref,
                 wm_ref, bm_ref, y_ref, p_ref):
    agg = a0_ref[0] + a1_ref[0]
    y = (
        jnp.dot(x_ref[...], wx_ref[...], preferred_element_type=jnp.float32)
        + jnp.dot(agg, wa_ref[...], preferred_element_type=jnp.float32)
        + bu_ref[...]
    )
    y_ref[...] = y
    p_ref[...] = (
        jnp.dot(y, wm_ref[...], preferred_element_type=jnp.float32)
        + bm_ref[...]
    )


def _update_and_project(x, aggp, wx, wa, bu, wm, bm):
    m, k = x.shape
    out = jax.ShapeDtypeStruct((m, k), jnp.float32)
    return pl.pallas_call(
        _update_body,
        grid=(1,),
        in_specs=[
            pl.BlockSpec((m, k), lambda i: (0, 0)),
            pl.BlockSpec((1, m, k), lambda i: (0, 0, 0)),
            pl.BlockSpec((1, m, k), lambda i: (1, 0, 0)),
            pl.BlockSpec((k, k), lambda i: (0, 0)),
            pl.BlockSpec((k, k), lambda i: (0, 0)),
            pl.BlockSpec((1, k), lambda i: (0, 0)),
            pl.BlockSpec((k, k), lambda i: (0, 0)),
            pl.BlockSpec((1, k), lambda i: (0, 0)),
        ],
        out_specs=[
            pl.BlockSpec((m, k), lambda i: (0, 0)),
            pl.BlockSpec((m, k), lambda i: (0, 0)),
        ],
        out_shape=[out, out],
    )(x, aggp, aggp, wx, wa, bu, wm, bm)


def _final_body(y_ref, a0_ref, a1_ref, oh0_ref, oh1_ref, oh2_ref, oh3_ref,
                wx_ref, wa_ref, bu_ref, wh_ref, bh_ref, wo_ref, bo_ref,
                o_ref):
    agg = a0_ref[0] + a1_ref[0]
    y2 = (
        jnp.dot(y_ref[...], wx_ref[...], preferred_element_type=jnp.float32)
        + jnp.dot(agg, wa_ref[...], preferred_element_type=jnp.float32)
        + bu_ref[...]
    )
    ohs = (oh0_ref, oh1_ref, oh2_ref, oh3_ref)
    iota64 = lax.broadcasted_iota(jnp.int32, (1, 64), 1)
    pooled = jnp.zeros((64, F), jnp.float32)
    for k in range(4):
        onehot = (ohs[k][...][:, None] == iota64).astype(jnp.float32)
        pooled += lax.dot_general(
            onehot, y2[:, k * F:(k + 1) * F],
            (((0,), (0,)), ((), ())), preferred_element_type=jnp.float32)
    h = jnp.maximum(
        jnp.dot(pooled, wh_ref[...], preferred_element_type=jnp.float32)
        + bh_ref[...], 0.0)
    o_ref[...] = (
        jnp.dot(h, wo_ref[...], preferred_element_type=jnp.float32)
        + bo_ref[...]
    )


def _final(y, aggp, oh, wx, wa, bu, wh, bh, wo, bo):
    m, k = y.shape
    return pl.pallas_call(
        _final_body,
        grid=(1,),
        in_specs=[
            pl.BlockSpec((m, k), lambda i: (0, 0)),
            pl.BlockSpec((1, m, k), lambda i: (0, 0, 0)),
            pl.BlockSpec((1, m, k), lambda i: (1, 0, 0)),
            pl.BlockSpec((m,), lambda i: (0,)),
            pl.BlockSpec((m,), lambda i: (0,)),
            pl.BlockSpec((m,), lambda i: (0,)),
            pl.BlockSpec((m,), lambda i: (0,)),
            pl.BlockSpec((k, k), lambda i: (0, 0)),
            pl.BlockSpec((k, k), lambda i: (0, 0)),
            pl.BlockSpec((1, k), lambda i: (0, 0)),
            pl.BlockSpec((F, F), lambda i: (0, 0)),
            pl.BlockSpec((1, F), lambda i: (0, 0)),
            pl.BlockSpec((F, 1), lambda i: (0, 0)),
            pl.BlockSpec((1, 1), lambda i: (0, 0)),
        ],
        out_specs=pl.BlockSpec((64, 1), lambda i: (0, 0)),
        out_shape=jax.ShapeDtypeStruct((64, 1), jnp.float32),
    )(y, aggp, aggp, oh[0], oh[1], oh[2], oh[3],
      wx, wa, bu, wh, bh, wo, bo)


def _make_edge_pass(n_nodes, n_pad, n_edges):
    n_sup = n_edges // SUP
    m_sup = -(-n_sup // NW)
    kloop = (m_sup + 5) // 3
    zb = 80
    n_zb = n_nodes // zb
    zmax = -(-n_zb // NS)

    mesh = plsc.VectorSubcoreMesh(core_axis_name="c", subcore_axis_name="s")

    @functools.partial(
        pl.kernel,
        out_type=jax.ShapeDtypeStruct((2 * n_pad, F), jnp.float32),
        mesh=mesh,
        scratch_types=[
            pltpu.VMEM((SUP,), jnp.int32),
            pltpu.VMEM((SUP,), jnp.int32),
            pltpu.VMEM((SUP,), jnp.int32),
            pltpu.VMEM((NSUB, CH), jnp.int32),
            pltpu.VMEM((NSUB, CH), jnp.int32),
            pltpu.VMEM((NSUB, CH), jnp.int32),
            pltpu.VMEM((SUP, F), jnp.float32),
            pltpu.VMEM((SUP, F), jnp.float32),
            pltpu.VMEM((SUP, F), jnp.float32),
            pltpu.VMEM((zb, F), jnp.float32),
            pltpu.VMEM_SHARED((n_nodes, F), jnp.float32),
            pltpu.SemaphoreType.DMA,
            pltpu.SemaphoreType.DMA,
            pltpu.SemaphoreType.DMA,
            pltpu.SemaphoreType.DMA,
            pltpu.SemaphoreType.DMA,
            pltpu.SemaphoreType.DMA,
            pltpu.SemaphoreType.DMA,
            pltpu.SemaphoreType.DMA,
            pltpu.SemaphoreType.DMA,
        ],
        compiler_params=pltpu.CompilerParams(use_tc_tiling_on_sc=False),
    )
    def edge_pass(p_hbm, q_hbm, src_hbm, dst_hbm, out_hbm,
                  sv0, sv1, sv2, dv0, dv1, dv2, gv0, gv1, gv2, z_v, acc_sh,
                  ls0, ls1, ls2, gs0, gs1, gs2, ss0, ss1, ss2):
        c = lax.axis_index("c")
        s = lax.axis_index("s")
        wid = s * NC + c
        sv = (sv0, sv1, sv2)
        dv = (dv0, dv1, dv2)
        gv = (gv0, gv1, gv2)
        ls = (ls0, ls1, ls2)
        gs = (gs0, gs1, gs2)
        ss = (ss0, ss1, ss2)

        def zfill(j, _):
            z_v[j, pl.ds(0, 16)] = jnp.zeros((16,), jnp.float32)
            z_v[j, pl.ds(16, 16)] = jnp.zeros((16,), jnp.float32)
            return 0
        lax.fori_loop(0, zb, zfill, 0)

        def zcopy(i, _):
            blk = s + i * NS

            @pl.when(blk < n_zb)
            def _():
                pltpu.sync_copy(z_v, acc_sh.at[pl.ds(blk * zb, zb)])
            return 0
        lax.fori_loop(0, zmax, zcopy, 0)
        plsc.subcore_barrier()

        def lin_cps(sup, b):
            base = sup * SUP
            cps = [
                pltpu.make_async_copy(
                    src_hbm.at[pl.ds(base, SUP)], sv[b], ls[b]),
                pltpu.make_async_copy(
                    q_hbm.at[pl.ds(base, SUP)], gv[b], ls[b]),
            ]
            for j in range(NSUB):
                cps.append(pltpu.make_async_copy(
                    dst_hbm.at[pl.ds(base + j * CH, CH)], dv[b].at[j], ls[b]))
            return cps

        def g_cps(b):
            return [
                pltpu.make_async_copy(
                    p_hbm.at[sv[b].at[pl.ds(j * CH, CH)]],
                    gv[b].at[pl.ds(j * CH, CH)], gs[b])
                for j in range(NSUB)
            ]

        def s_cps(b):
            return [
                pltpu.make_async_copy(
                    gv[b].at[pl.ds(j * CH, CH)],
                    acc_sh.at[dv[b].at[j]], ss[b])
                for j in range(NSUB)
            ]

        def stage(i, a):
            b = (a + 2) % 3
            nxt = (a + 1) % 3
            sup_a = wid + i * NW

            @pl.when(sup_a < n_sup)
            def _():
                for cp in lin_cps(sup_a, a):
                    cp.wait()
                for cp in g_cps(a):
                    cp.start(add=True)

            sup_b = wid + (i - 1) * NW

            @pl.when(jnp.logical_and(i >= 1, sup_b < n_sup))
            def _():
                for cp in g_cps(b):
                    cp.wait()

                def relu_body(j, _):
                    for h in range(2):
                        sl = pl.ds(h * 16, 16)
                        gv[b][j, sl] = jnp.maximum(gv[b][j, sl], 0.0)
                    return 0
                lax.fori_loop(0, SUP, relu_body, 0, unroll=8)
                for cp in s_cps(b):
                    cp.start(add=True)

            sup_d = wid + (i - 2) * NW

            @pl.when(jnp.logical_and(i >= 2, sup_d < n_sup))
            def _():
                for cp in s_cps(nxt):
                    cp.wait()

            sup_c = wid + (i + 1) * NW

            @pl.when(sup_c < n_sup)
            def _():
                for cp in lin_cps(sup_c, nxt):
                    cp.start()

        @pl.when(wid < n_sup)
        def _():
            for cp in lin_cps(wid, 0):
                cp.start()

        def loop_body(k, _):
            stage(3 * k, 0)
            stage(3 * k + 1, 1)
            stage(3 * k + 2, 2)
            return 0
        lax.fori_loop(0, kloop, loop_body, 0)
        plsc.subcore_barrier()

        if n_pad > n_nodes:
            @pl.when(s == 0)
            def _():
                pltpu.sync_copy(
                    z_v.at[pl.ds(0, n_pad - n_nodes)],
                    out_hbm.at[pl.ds(c * n_pad + n_nodes, n_pad - n_nodes)])

        def fcopy(i, _):
            blk = s + i * NS

            @pl.when(blk < n_zb)
            def _():
                row0 = blk * zb
                pltpu.sync_copy(acc_sh.at[pl.ds(row0, zb)], z_v)
                pltpu.sync_copy(z_v, out_hbm.at[pl.ds(c * n_pad + row0, zb)])
            return 0
        lax.fori_loop(0, zmax, fcopy, 0)

    return edge_pass


def kernel(node_features, edge_features, edge_idx, batch_idx,
           WM1, bM1, WU1, bU1, WM2, bM2, WU2, bU2, Wh, bh, Wo, bo):
    n = node_features.shape[0]
    n_pad = -(-n // 32) * 32
    m4 = n_pad // 4

    eye4 = jnp.eye(4, dtype=jnp.float32)

    def bd4(w):
        return jnp.kron(eye4, w)

    def tile4(b):
        return jnp.tile(b, 4)[None, :]

    xp = jnp.pad(node_features, ((0, n_pad - n), (0, F - 21)))
    x_packed = xp.reshape(m4, 4 * F)

    n_edges = edge_features.shape[0]
    src = edge_idx[0]
    dst = edge_idx[1]

    bfull = jnp.pad(batch_idx, (0, n_pad - n), constant_values=64)
    oh = tuple(bfull[k::4] for k in range(4))

    wm1x = jnp.pad(WM1[:21], ((0, 11), (0, 0)))
    wu1x = jnp.pad(WU1[:21], ((0, 11), (0, 0)))

    edge_pass = _make_edge_pass(n, n_pad, n_edges)

    p1 = _mm_bias(x_packed, bd4(wm1x), tile4(bM1))
    q1, q2 = _edge_proj(
        edge_features.T, WM1[21:24], WM2[32:35], block=16000)
    agg1 = edge_pass(p1.reshape(n_pad, F), q1, src, dst)
    y, p2 = _update_and_project(
        x_packed, agg1.reshape(2, m4, 4 * F), bd4(wu1x), bd4(WU1[21:53]),
        tile4(bU1), bd4(WM2[:32]), tile4(bM2))

    agg2 = edge_pass(p2.reshape(n_pad, F), q2, src, dst)

    return _final(
        y, agg2.reshape(2, m4, 4 * F), oh, bd4(WU2[:32]), bd4(WU2[32:64]),
        tile4(bU2), Wh, bh[None, :], Wo, bo[None, :])


# speedup vs baseline: 3.7933x; 1.3693x over previous
import functools

import jax
import jax.numpy as jnp
from jax import lax
from jax.experimental import pallas as pl
from jax.experimental.pallas import tpu as pltpu
from jax.experimental.pallas import tpu_sc as plsc

F = 32
NC = 2
NS = 16
NW = NC * NS
CH = 128
NSUB = 2
SUP = NSUB * CH


def _mm_bias_body(x_ref, w_ref, b_ref, o_ref):
    o_ref[...] = (
        jnp.dot(x_ref[...], w_ref[...], preferred_element_type=jnp.float32)
        + b_ref[...]
    )


def _mm_bias(x, w, b):
    m, k = x.shape
    f = w.shape[1]
    return pl.pallas_call(
        _mm_bias_body,
        grid=(1,),
        in_specs=[
            pl.BlockSpec((m, k), lambda i: (0, 0)),
            pl.BlockSpec((k, f), lambda i: (0, 0)),
            pl.BlockSpec((1, f), lambda i: (0, 0)),
        ],
        out_specs=pl.BlockSpec((m, f), lambda i: (0, 0)),
        out_shape=jax.ShapeDtypeStruct((m, f), jnp.float32),
    )(x, w, b)


def _edge_proj_body(et_ref, w_ref, q_ref):
    q_ref[...] = lax.dot_general(
        et_ref[...], w_ref[...], (((0,), (0,)), ((), ())),
        preferred_element_type=jnp.float32)


def _edge_proj(et, w, block):
    k, m = et.shape
    return pl.pallas_call(
        _edge_proj_body,
        grid=(m // block,),
        in_specs=[
            pl.BlockSpec((k, block), lambda i: (0, i)),
            pl.BlockSpec((k, F), lambda i: (0, 0)),
        ],
        out_specs=pl.BlockSpec((block, F), lambda i: (i, 0)),
        out_shape=jax.ShapeDtypeStruct((m, F), jnp.float32),
        compiler_params=pltpu.CompilerParams(
            fuse_transposed_lhs_in_matmul=True),
    )(et, w)


def _update_body(x_ref, a0_ref, a1_ref, wx_ref, wa_ref, bu_ref,
                 wm_ref, bm_ref, y_ref, p_ref):
    agg = a0_ref[0] + a1_ref[0]
    y = (
        jnp.dot(x_ref[...], wx_ref[...], preferred_element_type=jnp.float32)
        + jnp.dot(agg, wa_ref[...], preferred_element_type=jnp.float32)
        + bu_ref[...]
    )
    y_ref[...] = y
    p_ref[...] = (
        jnp.dot(y, wm_ref[...], preferred_element_type=jnp.float32)
        + bm_ref[...]
    )


def _update_and_project(x, aggp, wx, wa, bu, wm, bm):
    m, k = x.shape
    out = jax.ShapeDtypeStruct((m, k), jnp.float32)
    return pl.pallas_call(
        _update_body,
        grid=(1,),
        in_specs=[
            pl.BlockSpec((m, k), lambda i: (0, 0)),
            pl.BlockSpec((1, m, k), lambda i: (0, 0, 0)),
            pl.BlockSpec((1, m, k), lambda i: (1, 0, 0)),
            pl.BlockSpec((k, k), lambda i: (0, 0)),
            pl.BlockSpec((k, k), lambda i: (0, 0)),
            pl.BlockSpec((1, k), lambda i: (0, 0)),
            pl.BlockSpec((k, k), lambda i: (0, 0)),
            pl.BlockSpec((1, k), lambda i: (0, 0)),
        ],
        out_specs=[
            pl.BlockSpec((m, k), lambda i: (0, 0)),
            pl.BlockSpec((m, k), lambda i: (0, 0)),
        ],
        out_shape=[out, out],
    )(x, aggp, aggp, wx, wa, bu, wm, bm)


def _final_body(y_ref, a0_ref, a1_ref, oh0_ref, oh1_ref, oh2_ref, oh3_ref,
                wx_ref, wa_ref, bu_ref, wh_ref, bh_ref, wo_ref, bo_ref,
                o_ref):
    agg = a0_ref[0] + a1_ref[0]
    y2 = (
        jnp.dot(y_ref[...], wx_ref[...], preferred_element_type=jnp.float32)
        + jnp.dot(agg, wa_ref[...], preferred_element_type=jnp.float32)
        + bu_ref[...]
    )
    ohs = (oh0_ref, oh1_ref, oh2_ref, oh3_ref)
    iota64 = lax.broadcasted_iota(jnp.int32, (1, 64), 1)
    pooled = jnp.zeros((64, F), jnp.float32)
    for k in range(4):
        onehot = (ohs[k][...][:, None] == iota64).astype(jnp.float32)
        pooled += lax.dot_general(
            onehot, y2[:, k * F:(k + 1) * F],
            (((0,), (0,)), ((), ())), preferred_element_type=jnp.float32)
    h = jnp.maximum(
        jnp.dot(pooled, wh_ref[...], preferred_element_type=jnp.float32)
        + bh_ref[...], 0.0)
    o_ref[...] = (
        jnp.dot(h, wo_ref[...], preferred_element_type=jnp.float32)
        + bo_ref[...]
    )


def _final(y, aggp, oh, wx, wa, bu, wh, bh, wo, bo):
    m, k = y.shape
    return pl.pallas_call(
        _final_body,
        grid=(1,),
        in_specs=[
            pl.BlockSpec((m, k), lambda i: (0, 0)),
            pl.BlockSpec((1, m, k), lambda i: (0, 0, 0)),
            pl.BlockSpec((1, m, k), lambda i: (1, 0, 0)),
            pl.BlockSpec((m,), lambda i: (0,)),
            pl.BlockSpec((m,), lambda i: (0,)),
            pl.BlockSpec((m,), lambda i: (0,)),
            pl.BlockSpec((m,), lambda i: (0,)),
            pl.BlockSpec((k, k), lambda i: (0, 0)),
            pl.BlockSpec((k, k), lambda i: (0, 0)),
            pl.BlockSpec((1, k), lambda i: (0, 0)),
            pl.BlockSpec((F, F), lambda i: (0, 0)),
            pl.BlockSpec((1, F), lambda i: (0, 0)),
            pl.BlockSpec((F, 1), lambda i: (0, 0)),
            pl.BlockSpec((1, 1), lambda i: (0, 0)),
        ],
        out_specs=pl.BlockSpec((64, 1), lambda i: (0, 0)),
        out_shape=jax.ShapeDtypeStruct((64, 1), jnp.float32),
    )(y, aggp, aggp, oh[0], oh[1], oh[2], oh[3],
      wx, wa, bu, wh, bh, wo, bo)


def _make_edge_pass(n_nodes, n_pad, n_edges, inline):
    n_sup = n_edges // SUP
    m_sup = -(-n_sup // NW)
    kloop = (m_sup + 5) // 3
    zb = 40
    n_zb = n_nodes // zb
    zmax = -(-n_zb // NS)

    mesh = plsc.VectorSubcoreMesh(core_axis_name="c", subcore_axis_name="s")

    scratch = [
        pltpu.VMEM((SUP,), jnp.int32),
        pltpu.VMEM((SUP,), jnp.int32),
        pltpu.VMEM((SUP,), jnp.int32),
        pltpu.VMEM((NSUB, CH), jnp.int32),
        pltpu.VMEM((NSUB, CH), jnp.int32),
        pltpu.VMEM((NSUB, CH), jnp.int32),
        pltpu.VMEM((SUP, F), jnp.float32),
        pltpu.VMEM((SUP, F), jnp.float32),
        pltpu.VMEM((SUP, F), jnp.float32),
        pltpu.VMEM((zb, F), jnp.float32),
        pltpu.VMEM_SHARED((n_nodes, F), jnp.float32),
    ] + [pltpu.SemaphoreType.DMA] * 9
    if inline:
        scratch += [pltpu.VMEM((3, SUP), jnp.float32)] * 3
        scratch += [pltpu.VMEM((96,), jnp.float32)]

    @functools.partial(
        pl.kernel,
        out_type=jax.ShapeDtypeStruct((2 * n_pad, F), jnp.float32),
        mesh=mesh,
        scratch_types=scratch,
        compiler_params=pltpu.CompilerParams(use_tc_tiling_on_sc=False),
    )
    def edge_pass(*refs):
        if inline:
            (p_hbm, e0_hbm, e1_hbm, e2_hbm, w_hbm, src_hbm, dst_hbm, out_hbm,
             sv0, sv1, sv2, dv0, dv1, dv2, gv0, gv1, gv2, z_v, acc_sh,
             ls0, ls1, ls2, gs0, gs1, gs2, ss0, ss1, ss2,
             ev0, ev1, ev2, w_v) = refs
            ev = (ev0, ev1, ev2)
            e_hbm = (e0_hbm, e1_hbm, e2_hbm)
        else:
            (p_hbm, q_hbm, src_hbm, dst_hbm, out_hbm,
             sv0, sv1, sv2, dv0, dv1, dv2, gv0, gv1, gv2, z_v, acc_sh,
             ls0, ls1, ls2, gs0, gs1, gs2, ss0, ss1, ss2) = refs
        c = lax.axis_index("c")
        s = lax.axis_index("s")
        wid = s * NC + c
        sv = (sv0, sv1, sv2)
        dv = (dv0, dv1, dv2)
        gv = (gv0, gv1, gv2)
        ls = (ls0, ls1, ls2)
        gs = (gs0, gs1, gs2)
        ss = (ss0, ss1, ss2)

        if inline:
            pltpu.sync_copy(w_hbm, w_v)
            wvec = [[w_v[pl.ds(k * F + h * 16, 16)] for h in range(2)]
                    for k in range(3)]

        def zfill(j, _):
            z_v[j, pl.ds(0, 16)] = jnp.zeros((16,), jnp.float32)
            z_v[j, pl.ds(16, 16)] = jnp.zeros((16,), jnp.float32)
            return 0
        lax.fori_loop(0, zb, zfill, 0)

        def zcopy(i, _):
            blk = s + i * NS

            @pl.when(blk < n_zb)
            def _():
                pltpu.sync_copy(z_v, acc_sh.at[pl.ds(blk * zb, zb)])
            return 0
        lax.fori_loop(0, zmax, zcopy, 0)
        plsc.subcore_barrier()

        def lin_cps(sup, b):
            base = sup * SUP
            cps = [pltpu.make_async_copy(
                src_hbm.at[pl.ds(base, SUP)], sv[b], ls[b])]
            if inline:
                for k in range(3):
                    cps.append(pltpu.make_async_copy(
                        e_hbm[k].at[pl.ds(base, SUP)], ev[b].at[k], ls[b]))
            else:
                cps.append(pltpu.make_async_copy(
                    q_hbm.at[pl.ds(base, SUP)], gv[b], ls[b]))
            for j in range(NSUB):
                cps.append(pltpu.make_async_copy(
                    dst_hbm.at[pl.ds(base + j * CH, CH)], dv[b].at[j], ls[b]))
            return cps

        def g_cps(b):
            return [
                pltpu.make_async_copy(
                    p_hbm.at[sv[b].at[pl.ds(j * CH, CH)]],
                    gv[b].at[pl.ds(j * CH, CH)], gs[b])
                for j in range(NSUB)
            ]

        def s_cps(b):
            return [
                pltpu.make_async_copy(
                    gv[b].at[pl.ds(j * CH, CH)],
                    acc_sh.at[dv[b].at[j]], ss[b])
                for j in range(NSUB)
            ]

        def stage(i, a):
            b = (a + 2) % 3
            nxt = (a + 1) % 3
            sup_a = wid + i * NW

            @pl.when(sup_a < n_sup)
            def _():
                for cp in lin_cps(sup_a, a):
                    cp.wait()
                for cp in g_cps(a):
                    cp.start(add=not inline)

            sup_b = wid + (i - 1) * NW

            @pl.when(jnp.logical_and(i >= 1, sup_b < n_sup))
            def _():
                for cp in g_cps(b):
                    cp.wait()

                if inline:
                    def fuse_body(g, _):
                        base16 = g * 16
                        e0v = ev[b][0, pl.ds(base16, 16)]
                        e1v = ev[b][1, pl.ds(base16, 16)]
                        e2v = ev[b][2, pl.ds(base16, 16)]
                        for jj in range(16):
                            j = base16 + jj
                            for h in range(2):
                                q = (e0v[jj] * wvec[0][h]
                                     + e1v[jj] * wvec[1][h]
                                     + e2v[jj] * wvec[2][h])
                                sl = pl.ds(h * 16, 16)
                                gv[b][j, sl] = jnp.maximum(
                                    gv[b][j, sl] + q, 0.0)
                        return 0
                    lax.fori_loop(0, SUP // 16, fuse_body, 0)
                else:
                    def relu_body(j, _):
                        for h in range(2):
                            sl = pl.ds(h * 16, 16)
                            gv[b][j, sl] = jnp.maximum(gv[b][j, sl], 0.0)
                        return 0
                    lax.fori_loop(0, SUP, relu_body, 0, unroll=8)
                for cp in s_cps(b):
                    cp.start(add=True)

            sup_d = wid + (i - 2) * NW

            @pl.when(jnp.logical_and(i >= 2, sup_d < n_sup))
            def _():
                for cp in s_cps(nxt):
                    cp.wait()

            sup_c = wid + (i + 1) * NW

            @pl.when(sup_c < n_sup)
            def _():
                for cp in lin_cps(sup_c, nxt):
                    cp.start()

        @pl.when(wid < n_sup)
        def _():
            for cp in lin_cps(wid, 0):
                cp.start()

        def loop_body(k, _):
            stage(3 * k, 0)
            stage(3 * k + 1, 1)
            stage(3 * k + 2, 2)
            return 0
        lax.fori_loop(0, kloop, loop_body, 0)
        plsc.subcore_barrier()

        if n_pad > n_nodes:
            @pl.when(s == 0)
            def _():
                pltpu.sync_copy(
                    z_v.at[pl.ds(0, n_pad - n_nodes)],
                    out_hbm.at[pl.ds(c * n_pad + n_nodes, n_pad - n_nodes)])

        def fcopy(i, _):
            blk = s + i * NS

            @pl.when(blk < n_zb)
            def _():
                row0 = blk * zb
                pltpu.sync_copy(acc_sh.at[pl.ds(row0, zb)], z_v)
                pltpu.sync_copy(z_v, out_hbm.at[pl.ds(c * n_pad + row0, zb)])
            return 0
        lax.fori_loop(0, zmax, fcopy, 0)

    return edge_pass


def kernel(node_features, edge_features, edge_idx, batch_idx,
           WM1, bM1, WU1, bU1, WM2, bM2, WU2, bU2, Wh, bh, Wo, bo):
    n = node_features.shape[0]
    n_pad = -(-n // 32) * 32
    m4 = n_pad // 4

    eye4 = jnp.eye(4, dtype=jnp.float32)

    def bd4(w):
        return jnp.kron(eye4, w)

    def tile4(b):
        return jnp.tile(b, 4)[None, :]

    xp = jnp.pad(node_features, ((0, n_pad - n), (0, F - 21)))
    x_packed = xp.reshape(m4, 4 * F)

    n_edges = edge_features.shape[0]
    src = edge_idx[0]
    dst = edge_idx[1]

    bfull = jnp.pad(batch_idx, (0, n_pad - n), constant_values=64)
    oh = tuple(bfull[k::4] for k in range(4))

    wm1x = jnp.pad(WM1[:21], ((0, 11), (0, 0)))
    wu1x = jnp.pad(WU1[:21], ((0, 11), (0, 0)))

    edge_pass1 = _make_edge_pass(n, n_pad, n_edges, inline=True)
    edge_pass2 = _make_edge_pass(n, n_pad, n_edges, inline=False)

    p1 = _mm_bias(x_packed, bd4(wm1x), tile4(bM1))
    agg1 = edge_pass1(
        p1.reshape(n_pad, F), edge_features[:, 0], edge_features[:, 1],
        edge_features[:, 2], WM1[21:24].reshape(96), src, dst)
    q2 = _edge_proj(edge_features.T, WM2[32:35], block=16000)
    y, p2 = _update_and_project(
        x_packed, agg1.reshape(2, m4, 4 * F), bd4(wu1x), bd4(WU1[21:53]),
        tile4(bU1), bd4(WM2[:32]), tile4(bM2))

    agg2 = edge_pass2(p2.reshape(n_pad, F), q2, src, dst)

    return _final(
        y, agg2.reshape(2, m4, 4 * F), oh, bd4(WU2[:32]), bd4(WU2[32:64]),
        tile4(bU2), Wh, bh[None, :], Wo, bo[None, :])
